# baseline (speedup 1.0000x reference)
import jax
import jax.numpy as jnp
from jax.experimental import pallas as pl
from jax.experimental.pallas import tpu as pltpu

N_ATOMS = 16384
N_STRUCT = 16
N_ELEM = 95
EMB = 64
HID = 256
XDIM = 128

BLOCK = 16384
HALF = 2048

_F32 = jnp.float32
_BF16 = jnp.bfloat16


def _hi_lo(a):
    hi = a.astype(_BF16)
    lo = (a - hi.astype(_F32)).astype(_BF16)
    return hi, lo


def _fused_body(coord_ref, elems_ref, ind_ref, emb_ref, w1c_ref, b1_ref,
                w2_ref, b2_ref, w3_ref, b3_ref, out_ref, ws_ref):
    i = pl.program_id(0)

    @pl.when(i == 0)
    def _init():
        m = jnp.dot(emb_ref[...], w1c_ref[0:EMB, :],
                    preferred_element_type=_F32)
        mh, ml = _hi_lo(m)
        ws_ref[...] = jnp.concatenate([
            mh,
            w1c_ref[EMB:, :].astype(_BF16),
            b1_ref[...].astype(_BF16),
            jnp.zeros((XDIM - N_ELEM - 4, HID), _BF16),
            ml,
            jnp.zeros((XDIM - N_ELEM, HID), _BF16),
        ], axis=0)

    ws = ws_ref[...]
    b2 = b2_ref[0, :]
    w2 = w2_ref[...]
    w3t = w3_ref[...]
    b3 = b3_ref[0, 0]

    parts = []
    for h in range(BLOCK // HALF):
        sl = pl.ds(h * HALF, HALF)
        elems = elems_ref[0, 0, sl]
        onehot = (jax.lax.broadcasted_iota(jnp.int32, (HALF, N_ELEM), 1)
                  == elems[:, None]).astype(_F32)
        x = jnp.concatenate(
            [onehot, coord_ref[sl, :], jnp.ones((HALF, 1), _F32),
             jnp.zeros((HALF, XDIM - N_ELEM - 4), _F32)], axis=1)
        xb = x.astype(_BF16)
        x2 = jnp.concatenate([xb, xb], axis=1)
        h1 = jnp.tanh(jnp.dot(x2, ws, preferred_element_type=_F32))
        h2 = jnp.tanh(jnp.dot(h1, w2, preferred_element_type=_F32) + b2)
        pa16 = jnp.dot(h2, w3t, preferred_element_type=_F32) + b3
        ind = ind_ref[0, 0, sl]
        seg = (jax.lax.broadcasted_iota(jnp.int32, (HALF, N_STRUCT), 1)
               == ind[:, None]).astype(_F32)
        parts.append(jnp.sum(seg * pa16, axis=0)[None, :])
    part = parts[0]
    for p in parts[1:]:
        part = part + p

    @pl.when(i == 0)
    def _first():
        out_ref[...] = part

    @pl.when(i != 0)
    def _acc():
        out_ref[...] += part


@jax.jit
def kernel(coord, elems, ind_1, elem_embed, W1, b1, W2, b2, W3, b3):
    n = coord.shape[0]
    grid = n // BLOCK
    elems3 = elems.astype(jnp.int32).reshape(grid, 1, BLOCK)
    ind3 = ind_1.astype(jnp.int32).reshape(grid, 1, BLOCK)
    w3t = jnp.tile(W3, (1, N_STRUCT))

    out = pl.pallas_call(
        _fused_body,
        grid=(grid,),
        in_specs=[
            pl.BlockSpec((BLOCK, 3), lambda i: (i, 0)),
            pl.BlockSpec((1, 1, BLOCK), lambda i: (i, 0, 0)),
            pl.BlockSpec((1, 1, BLOCK), lambda i: (i, 0, 0)),
            pl.BlockSpec((N_ELEM, EMB), lambda i: (0, 0)),
            pl.BlockSpec((EMB + 3, HID), lambda i: (0, 0)),
            pl.BlockSpec((1, HID), lambda i: (0, 0)),
            pl.BlockSpec((HID, HID), lambda i: (0, 0)),
            pl.BlockSpec((1, HID), lambda i: (0, 0)),
            pl.BlockSpec((HID, N_STRUCT), lambda i: (0, 0)),
            pl.BlockSpec((1, 1), lambda i: (0, 0)),
        ],
        out_specs=pl.BlockSpec((1, N_STRUCT), lambda i: (0, 0)),
        out_shape=jax.ShapeDtypeStruct((1, N_STRUCT), jnp.float32),
        scratch_shapes=[
            pltpu.VMEM((2 * XDIM, HID), _BF16),
        ],
    )(coord, elems3, ind3, elem_embed, W1, b1.reshape(1, HID), W2,
      b2.reshape(1, HID), w3t, b3.reshape(1, 1))
    return out[0]


# speedup vs baseline: 1.0209x; 1.0209x over previous
import jax
import jax.numpy as jnp
from jax.experimental import pallas as pl
from jax.experimental.pallas import tpu as pltpu

N_ATOMS = 16384
N_STRUCT = 16
N_ELEM = 95
EMB = 64
HID = 256
XDIM = 128

BLOCK = 8192
HALF = 1024

_F32 = jnp.float32
_BF16 = jnp.bfloat16


def _hi_lo(a):
    hi = a.astype(_BF16)
    lo = (a - hi.astype(_F32)).astype(_BF16)
    return hi, lo


def _fused_body(coord_ref, elems_ref, ind_ref, emb_ref, w1c_ref, b1_ref,
                w2_ref, b2_ref, w3_ref, b3_ref, out_ref, ws_ref):
    i = pl.program_id(0)

    @pl.when(i == 0)
    def _init():
        m = jnp.dot(emb_ref[...], w1c_ref[0:EMB, :],
                    preferred_element_type=_F32)
        mh, ml = _hi_lo(m)
        ws_ref[...] = jnp.concatenate([
            mh,
            w1c_ref[EMB:, :].astype(_BF16),
            b1_ref[...].astype(_BF16),
            jnp.zeros((XDIM - N_ELEM - 4, HID), _BF16),
            ml,
            jnp.zeros((XDIM - N_ELEM, HID), _BF16),
        ], axis=0)

    ws = ws_ref[...]
    b2 = b2_ref[0, :]
    w2 = w2_ref[...]
    w3t = w3_ref[...]
    b3 = b3_ref[0, 0]

    parts = []
    for h in range(BLOCK // HALF):
        sl = pl.ds(h * HALF, HALF)
        elems = elems_ref[0, 0, sl]
        onehot = (jax.lax.broadcasted_iota(jnp.int32, (HALF, N_ELEM), 1)
                  == elems[:, None]).astype(_F32)
        x = jnp.concatenate(
            [onehot, coord_ref[sl, :], jnp.ones((HALF, 1), _F32),
             jnp.zeros((HALF, XDIM - N_ELEM - 4), _F32)], axis=1)
        xb = x.astype(_BF16)
        x2 = jnp.concatenate([xb, xb], axis=1)
        h1 = jnp.tanh(jnp.dot(x2, ws, preferred_element_type=_F32))
        h2 = jnp.tanh(jnp.dot(h1, w2, preferred_element_type=_F32) + b2)
        pa16 = jnp.dot(h2, w3t, preferred_element_type=_F32) + b3
        ind = ind_ref[0, 0, sl]
        seg = (jax.lax.broadcasted_iota(jnp.int32, (HALF, N_STRUCT), 1)
               == ind[:, None]).astype(_F32)
        parts.append(jnp.sum(seg * pa16, axis=0)[None, :])
    part = parts[0]
    for p in parts[1:]:
        part = part + p

    @pl.when(i == 0)
    def _first():
        out_ref[...] = part

    @pl.when(i != 0)
    def _acc():
        out_ref[...] += part


@jax.jit
def kernel(coord, elems, ind_1, elem_embed, W1, b1, W2, b2, W3, b3):
    n = coord.shape[0]
    grid = n // BLOCK
    elems3 = elems.astype(jnp.int32).reshape(grid, 1, BLOCK)
    ind3 = ind_1.astype(jnp.int32).reshape(grid, 1, BLOCK)
    w3t = jnp.tile(W3, (1, N_STRUCT))

    out = pl.pallas_call(
        _fused_body,
        grid=(grid,),
        in_specs=[
            pl.BlockSpec((BLOCK, 3), lambda i: (i, 0)),
            pl.BlockSpec((1, 1, BLOCK), lambda i: (i, 0, 0)),
            pl.BlockSpec((1, 1, BLOCK), lambda i: (i, 0, 0)),
            pl.BlockSpec((N_ELEM, EMB), lambda i: (0, 0)),
            pl.BlockSpec((EMB + 3, HID), lambda i: (0, 0)),
            pl.BlockSpec((1, HID), lambda i: (0, 0)),
            pl.BlockSpec((HID, HID), lambda i: (0, 0)),
            pl.BlockSpec((1, HID), lambda i: (0, 0)),
            pl.BlockSpec((HID, N_STRUCT), lambda i: (0, 0)),
            pl.BlockSpec((1, 1), lambda i: (0, 0)),
        ],
        out_specs=pl.BlockSpec((1, N_STRUCT), lambda i: (0, 0)),
        out_shape=jax.ShapeDtypeStruct((1, N_STRUCT), jnp.float32),
        scratch_shapes=[
            pltpu.VMEM((2 * XDIM, HID), _BF16),
        ],
    )(coord, elems3, ind3, elem_embed, W1, b1.reshape(1, HID), W2,
      b2.reshape(1, HID), w3t, b3.reshape(1, 1))
    return out[0]
